# baseline (speedup 1.0000x reference)
import functools

import numpy as np
import jax
import jax.numpy as jnp
from jax.experimental import pallas as pl
from jax.experimental.pallas import tpu as pltpu

_VGG19_STD = np.asarray([0.229, 0.224, 0.225], dtype=np.float32)
_W_C = (np.float32(0.25) / (_VGG19_STD * _VGG19_STD)).astype(np.float32)


def _wsq_fold_kernel(x_ref, y_ref, o_ref, *, bm, lanes, sub, w0, w1, w2):
    i = pl.program_id(0)
    d = x_ref[...] - y_ref[...]
    c2 = d * d
    s = c2[:, 0:128]
    for j in range(1, lanes // 128):
        s = s + c2[:, j * 128:(j + 1) * 128]
    r = jax.lax.broadcasted_iota(jnp.int32, (bm, 1), 0) + i * bm
    c = (r // sub) % 3
    w = jnp.where(c == 0, w0, jnp.where(c == 1, w1, w2))
    s = s * w
    nfull, rem = bm // 8, bm % 8
    if nfull:
        t = s[0:8, :]
        for j in range(1, nfull):
            t = t + s[j * 8:(j + 1) * 8, :]
    if rem:
        tail = jnp.concatenate(
            [s[nfull * 8:bm, :], jnp.zeros((8 - rem, 128), jnp.float32)], axis=0)
        t = t + tail if nfull else tail
    o_ref[...] = t


def _weighted_partials(x2, y2, sub, w0, w1, w2):
    m2, lanes = x2.shape
    bm = m2
    for cand in (96, 192, 48, 24, 8):
        if m2 % cand == 0 and (m2 // cand) % 2 == 0:
            bm = cand
            break
    grid = m2 // bm

    block_in = bm * lanes * 4
    vmem_limit = int(min(2 * 2 * block_in + (4 << 20), 60 << 20))

    body = functools.partial(_wsq_fold_kernel, bm=bm, lanes=lanes,
                             sub=sub, w0=w0, w1=w1, w2=w2)
    out = pl.pallas_call(
        body,
        out_shape=jax.ShapeDtypeStruct((grid * 8, 128), jnp.float32),
        grid=(grid,),
        in_specs=[
            pl.BlockSpec((bm, lanes), lambda i: (i, 0)),
            pl.BlockSpec((bm, lanes), lambda i: (i, 0)),
        ],
        out_specs=pl.BlockSpec((8, 128), lambda i: (i, 0)),
        compiler_params=pltpu.CompilerParams(
            dimension_semantics=("parallel",),
            vmem_limit_bytes=vmem_limit,
        ),
        cost_estimate=pl.CostEstimate(
            flops=3 * m2 * lanes,
            transcendentals=0,
            bytes_accessed=2 * m2 * lanes * 4 + grid * 8 * 128 * 4,
        ),
    )(x2, y2)
    return out


def kernel(x, y):
    n, c_in, h, w = x.shape
    hw = h * w

    if c_in == 3:
        w0, w1, w2 = float(_W_C[0]), float(_W_C[1]), float(_W_C[2])
    else:
        ws = float(np.float32(_W_C[0] + _W_C[1] + _W_C[2]))
        w0 = w1 = w2 = ws

    lanes = hw
    for cand in (8192, 4096, 2048, 1024, 512, 256, 128):
        if hw >= cand and hw % cand == 0:
            lanes = cand
            break
    sub = hw // lanes

    m2 = n * c_in * sub
    x2 = x.reshape(m2, lanes)
    y2 = y.reshape(m2, lanes)

    partials = _weighted_partials(x2, y2, sub, w0, w1, w2)
    scale = np.float32(1.0) / (np.float32(3.0) * np.float32(hw) * np.float32(n))
    return jnp.sum(partials) * scale


# speedup vs baseline: 4.3075x; 4.3075x over previous
import functools

import numpy as np
import jax
import jax.numpy as jnp
from jax.experimental import pallas as pl
from jax.experimental.pallas import tpu as pltpu

_VGG19_STD = np.asarray([0.229, 0.224, 0.225], dtype=np.float32)
_W_C = (np.float32(0.25) / (_VGG19_STD * _VGG19_STD)).astype(np.float32)


def _fold_hw(v, h, w):
    s = v[..., 0:128]
    for j in range(1, w // 128):
        s = s + v[..., j * 128:(j + 1) * 128]
    t = s[:, 0:8, :]
    for j in range(1, h // 8):
        t = t + s[:, j * 8:(j + 1) * 8, :]
    u = t[0]
    for j in range(1, t.shape[0]):
        u = u + t[j]
    return u


def _wsq_kernel(x_ref, y_ref, o_ref, *, h, w, weights):
    d = x_ref[...] - y_ref[...]
    c2 = d * d
    acc = None
    for c, wc in enumerate(weights):
        part = _fold_hw(c2[:, c], h, w) * wc
        acc = part if acc is None else acc + part
    o_ref[...] = acc


def kernel(x, y):
    n, c_in, h, w = x.shape

    if c_in == 3:
        weights = (float(_W_C[0]), float(_W_C[1]), float(_W_C[2]))
    else:
        weights = (float(np.float32(_W_C[0] + _W_C[1] + _W_C[2])),)

    bn = 1
    for cand in (4, 2, 1):
        if n % cand == 0 and (n // cand) % 2 == 0:
            bn = cand
            break
    grid = n // bn

    block_in = bn * c_in * h * w * 4
    vmem_limit = int(min(2 * 2 * block_in + (4 << 20), 60 << 20))

    body = functools.partial(_wsq_kernel, h=h, w=w, weights=weights)
    partials = pl.pallas_call(
        body,
        out_shape=jax.ShapeDtypeStruct((grid * 8, 128), jnp.float32),
        grid=(grid,),
        in_specs=[
            pl.BlockSpec((bn, c_in, h, w), lambda i: (i, 0, 0, 0)),
            pl.BlockSpec((bn, c_in, h, w), lambda i: (i, 0, 0, 0)),
        ],
        out_specs=pl.BlockSpec((8, 128), lambda i: (i, 0)),
        compiler_params=pltpu.CompilerParams(
            dimension_semantics=("parallel",),
            vmem_limit_bytes=vmem_limit,
        ),
        cost_estimate=pl.CostEstimate(
            flops=3 * n * c_in * h * w,
            transcendentals=0,
            bytes_accessed=2 * n * c_in * h * w * 4 + grid * 8 * 128 * 4,
        ),
    )(x, y)

    scale = np.float32(1.0) / (np.float32(3.0) * np.float32(h * w) * np.float32(n))
    return jnp.sum(partials) * scale
